# baseline (speedup 1.0000x reference)
import functools

import jax
import jax.numpy as jnp
from jax import lax
from jax.experimental import pallas as pl
from jax.experimental.pallas import tpu as pltpu
from jax.experimental.pallas import tpu_sc as plsc

N = 3276800
NC, NS, L = 2, 16, 16
NW = NC * NS
PW = N // NW
CH = 12800
NCH = PW // CH
TBL = 32

_mesh = plsc.VectorSubcoreMesh(
    core_axis_name="c", subcore_axis_name="s", num_cores=NC, num_subcores=NS
)


@functools.partial(
    pl.kernel,
    out_type=jax.ShapeDtypeStruct((N,), jnp.float32),
    mesh=_mesh,
    scratch_types=[
        pltpu.VMEM((TBL,), jnp.float32),
        pltpu.VMEM((CH,), jnp.int32),
        pltpu.VMEM((CH,), jnp.float32),
    ],
    compiler_params=pltpu.CompilerParams(needs_layout_passes=False),
)
def _lookup(idx_hbm, table_hbm, out_hbm, table_v, idx_v, out_v):
    wid = lax.axis_index("s") * NC + lax.axis_index("c")
    base = wid * PW
    pltpu.sync_copy(table_hbm, table_v)

    def chunk_body(g, carry):
        off = base + g * CH
        pltpu.sync_copy(idx_hbm.at[pl.ds(off, CH)], idx_v)

        def vec_body(i, c):
            idx = idx_v[pl.ds(i * L, L)]
            out_v[pl.ds(i * L, L)] = plsc.load_gather(table_v, [idx])
            return c

        lax.fori_loop(0, CH // L, vec_body, 0)
        pltpu.sync_copy(out_v, out_hbm.at[pl.ds(off, CH)])
        return carry

    lax.fori_loop(0, NCH, chunk_body, 0)


def kernel(node_gt, mapping_tensor):
    return _lookup(node_gt, mapping_tensor)


# speedup vs baseline: 2.0002x; 2.0002x over previous
import functools

import jax
import jax.numpy as jnp
from jax import lax
from jax.experimental import pallas as pl
from jax.experimental.pallas import tpu as pltpu
from jax.experimental.pallas import tpu_sc as plsc

N = 3276800
NC, NS, L = 2, 16, 16
NW = NC * NS
PW = N // NW
CH = 12800
NCH = PW // CH
NBUF = 2
UNROLL = 8
TBL = 32

_mesh = plsc.VectorSubcoreMesh(
    core_axis_name="c", subcore_axis_name="s", num_cores=NC, num_subcores=NS
)


@functools.partial(
    pl.kernel,
    out_type=jax.ShapeDtypeStruct((N,), jnp.float32),
    mesh=_mesh,
    scratch_types=[
        pltpu.VMEM((TBL,), jnp.float32),
        pltpu.VMEM((NBUF, CH), jnp.int32),
        pltpu.VMEM((NBUF, CH), jnp.float32),
        pltpu.SemaphoreType.DMA,
        pltpu.SemaphoreType.DMA,
        pltpu.SemaphoreType.DMA,
        pltpu.SemaphoreType.DMA,
    ],
    compiler_params=pltpu.CompilerParams(needs_layout_passes=False),
)
def _lookup(idx_hbm, table_hbm, out_hbm, table_v, idx_v, out_v,
            in_s0, in_s1, out_s0, out_s1):
    wid = lax.axis_index("s") * NC + lax.axis_index("c")
    base = wid * PW
    in_sem = (in_s0, in_s1)
    out_sem = (out_s0, out_s1)
    pltpu.sync_copy(table_hbm, table_v)

    def in_slice(g):
        return idx_hbm.at[pl.ds(base + g * CH, CH)]

    def out_slice(g):
        return out_hbm.at[pl.ds(base + g * CH, CH)]

    loads = {}
    stores = {}
    for g in range(NBUF):
        loads[g] = pltpu.async_copy(in_slice(g), idx_v.at[g % NBUF], in_sem[g % NBUF])
    for g in range(NCH):
        b = g % NBUF
        loads[g].wait()
        if g >= NBUF:
            stores[g - NBUF].wait()

        @plsc.parallel_loop(0, CH, step=L, unroll=UNROLL)
        def _gather(i):
            out_v[b, pl.ds(i, L)] = plsc.load_gather(
                table_v, [idx_v[b, pl.ds(i, L)]]
            )

        stores[g] = pltpu.async_copy(out_v.at[b], out_slice(g), out_sem[b])
        if g + NBUF < NCH:
            loads[g + NBUF] = pltpu.async_copy(
                in_slice(g + NBUF), idx_v.at[b], in_sem[b]
            )
    for g in range(NCH - NBUF, NCH):
        stores[g].wait()


def kernel(node_gt, mapping_tensor):
    return _lookup(node_gt, mapping_tensor)
